# baseline (speedup 1.0000x reference)
import functools

import jax
import jax.numpy as jnp
from jax import lax
from jax.experimental import pallas as pl
from jax.experimental.pallas import tpu as pltpu
from jax.experimental.pallas import tpu_sc as plsc

_NUM_HEADS = 16
_MAX_DIST = 128
_S = 2048
_HALF = _S // 2
_NSHIFT = 8
_UM = 3200
_L = 16
_GROUP = 16
_GGROUP = 8


def _sc_body(table_hbm, out_hbm, idx_v, m_v, sem):
    h = lax.axis_index("s")
    half = lax.axis_index("c")
    r0 = half * _HALF
    b_base = r0 + _HALF + 6

    lane = lax.iota(jnp.int32, _L)

    def build_idx(c, _):
        u0 = c * _L
        for s in range(_NSHIFT):
            d = u0 + lane - b_base + s
            idx = (jnp.clip(d, -_MAX_DIST, _MAX_DIST) + _MAX_DIST) * _NUM_HEADS + h
            idx_v[pl.ds(pl.multiple_of(s * _UM + u0, _L), _L)] = idx
        return _

    lax.fori_loop(0, _UM // _L, build_idx, None)

    def gather_group(q, _):
        for k in range(_GGROUP):
            off = pl.multiple_of((q * _GGROUP + k) * 128, 128)
            pltpu.async_copy(
                table_hbm.at[idx_v.at[pl.ds(off, 128)]],
                m_v.at[pl.ds(off, 128)],
                sem,
            )
        return _

    n_gather = _NSHIFT * _UM // 128
    lax.fori_loop(0, n_gather // _GGROUP, gather_group, None)

    def drain_gather(q, _):
        for _k in range(_GGROUP):
            pltpu.make_async_copy(
                table_hbm.at[pl.ds(0, 128)], m_v.at[pl.ds(0, 128)], sem
            ).wait()
        return _

    lax.fori_loop(0, n_gather // _GGROUP, drain_gather, None)

    def row_group(g, _):
        base = r0 + g * _GROUP
        for k in range(_GROUP):
            i = base + k
            t = b_base - i
            s = lax.rem(t, _NSHIFT)
            o = pl.multiple_of(s * _UM + (t - s), _NSHIFT)
            dst = pl.multiple_of((h * _S + i) * _S, _S)
            pltpu.async_copy(
                m_v.at[pl.ds(o, _S)], out_hbm.at[pl.ds(dst, _S)], sem
            )
        return _

    lax.fori_loop(0, _HALF // _GROUP, row_group, None)

    def drain_rows(g, _):
        for _k in range(_GROUP):
            pltpu.make_async_copy(
                out_hbm.at[pl.ds(0, _S)], m_v.at[pl.ds(0, _S)], sem
            ).wait()
        return _

    lax.fori_loop(0, _HALF // _GROUP, drain_rows, None)


@functools.partial(
    pl.kernel,
    mesh=plsc.VectorSubcoreMesh(core_axis_name="c", subcore_axis_name="s"),
    out_type=jax.ShapeDtypeStruct((_NUM_HEADS * _S * _S,), jnp.float32),
    scratch_types=[
        pltpu.VMEM((_NSHIFT * _UM,), jnp.int32),
        pltpu.VMEM((_NSHIFT * _UM,), jnp.float32),
        pltpu.SemaphoreType.DMA,
    ],
)
def _sc_rel_pos(table_hbm, out_hbm, idx_v, m_v, sem):
    _sc_body(table_hbm, out_hbm, idx_v, m_v, sem)


def kernel(seq_len, table):
    del seq_len
    out = _sc_rel_pos(table.reshape(-1))
    return out.reshape(_NUM_HEADS, _S, _S)


# speedup vs baseline: 4.8042x; 4.8042x over previous
import functools

import jax
import jax.numpy as jnp
from jax import lax
from jax.experimental import pallas as pl
from jax.experimental.pallas import tpu as pltpu
from jax.experimental.pallas import tpu_sc as plsc

_NUM_HEADS = 16
_MAX_DIST = 128
_S = 2048
_L = 16
_BT = 256
_NB = _S // _BT
_WW = 2 * _BT
_NDELTA = 3
_HW = _NDELTA * _WW
_PW = _HW // 2


def _sc_bands_body(table_hbm, bands_hbm, idx_v, seg_v, sem):
    h = lax.axis_index("s")
    half = lax.axis_index("c")
    p0 = half * _PW

    lane = lax.iota(jnp.int32, _L)

    def build_idx(ch, _):
        p = p0 + ch * _L + lane
        k = lax.shift_right_logical(p, 9)
        t = lax.bitwise_and(p, _WW - 1)
        delta = (lax.bitwise_and(k, 1) - lax.shift_right_logical(k, 1)) * _BT
        d = t - (_BT - 1) + delta
        idx = (jnp.clip(d, -_MAX_DIST, _MAX_DIST) + _MAX_DIST) * _NUM_HEADS + h
        idx_v[pl.ds(pl.multiple_of(ch * _L, _L), _L)] = idx
        return _

    lax.fori_loop(0, _PW // _L, build_idx, None)

    for q in range(_PW // 128):
        off = 128 * q
        pltpu.async_copy(
            table_hbm.at[idx_v.at[pl.ds(off, 128)]],
            seg_v.at[pl.ds(off, 128)],
            sem,
        )
    for q in range(_PW // 128):
        pltpu.make_async_copy(
            table_hbm.at[pl.ds(0, 128)], seg_v.at[pl.ds(0, 128)], sem
        ).wait()

    woff = pl.multiple_of(h * _HW + p0, _L)
    pltpu.sync_copy(seg_v, bands_hbm.at[pl.ds(woff, _PW)])


@functools.partial(
    pl.kernel,
    mesh=plsc.VectorSubcoreMesh(core_axis_name="c", subcore_axis_name="s"),
    out_type=jax.ShapeDtypeStruct((_NUM_HEADS * _HW,), jnp.float32),
    scratch_types=[
        pltpu.VMEM((_PW,), jnp.int32),
        pltpu.VMEM((_PW,), jnp.float32),
        pltpu.SemaphoreType.DMA,
    ],
)
def _sc_bands(table_hbm, bands_hbm, idx_v, seg_v, sem):
    _sc_bands_body(table_hbm, bands_hbm, idx_v, seg_v, sem)


def _shear_tile(w_row):
    y = jnp.broadcast_to(w_row[None, :], (_BT, _WW))
    row = lax.broadcasted_iota(jnp.int32, (_BT, _WW), 0)
    for b in range(8):
        amt = 1 << b
        shifted = jnp.concatenate(
            [y[:, amt:], jnp.zeros((_BT, amt), jnp.float32)], axis=1
        )
        apply = ((row >> b) & 1) == 0
        y = jnp.where(apply, shifted, y)
    return y[:, :_BT]


def _tc_body(bands_ref, out_ref, scr_ref):
    ib = pl.program_id(1)
    jb = pl.program_id(2)

    @pl.when((ib == 0) & (jb == 0))
    def _build():
        for k in range(_NDELTA):
            scr_ref[k] = _shear_tile(bands_ref[0, k, :])

    db = jb - ib

    @pl.when(db == 0)
    def _d0():
        out_ref[0] = scr_ref[0]

    @pl.when(db == 1)
    def _dp():
        out_ref[0] = scr_ref[1]

    @pl.when(db == -1)
    def _dm():
        out_ref[0] = scr_ref[2]

    @pl.when(db >= 2)
    def _hi():
        out_ref[0] = jnp.broadcast_to(scr_ref[1, 0:1, 255:256], (_BT, _BT))

    @pl.when(db <= -2)
    def _lo():
        out_ref[0] = jnp.broadcast_to(scr_ref[2, 0:1, 0:1], (_BT, _BT))


@functools.partial(
    pl.pallas_call,
    grid=(_NUM_HEADS, _NB, _NB),
    in_specs=[
        pl.BlockSpec((1, _NDELTA, _WW), lambda hh, ib, jb: (hh, 0, 0)),
    ],
    out_specs=pl.BlockSpec((1, _BT, _BT), lambda hh, ib, jb: (hh, ib, jb)),
    out_shape=jax.ShapeDtypeStruct((_NUM_HEADS, _S, _S), jnp.float32),
    scratch_shapes=[pltpu.VMEM((_NDELTA, _BT, _BT), jnp.float32)],
)
def _tc_expand(bands_ref, out_ref, scr_ref):
    _tc_body(bands_ref, out_ref, scr_ref)


def kernel(seq_len, table):
    del seq_len
    bands = _sc_bands(table.reshape(-1))
    return _tc_expand(bands.reshape(_NUM_HEADS, _NDELTA, _WW))


# speedup vs baseline: 10.8376x; 2.2558x over previous
import functools

import jax
import jax.numpy as jnp
from jax import lax
from jax.experimental import pallas as pl
from jax.experimental.pallas import tpu as pltpu
from jax.experimental.pallas import tpu_sc as plsc

_NUM_HEADS = 16
_MAX_DIST = 128
_S = 2048
_L = 16
_BT = 256
_NB = _S // _BT
_WW = 2 * _BT
_NDELTA = 3
_HW = _NDELTA * _WW
_PW = _HW // 2


def _sc_bands_body(table_hbm, bands_hbm, idx_v, seg_v, sem):
    h = lax.axis_index("s")
    half = lax.axis_index("c")
    p0 = half * _PW

    lane = lax.iota(jnp.int32, _L)

    def build_idx(ch, _):
        p = p0 + ch * _L + lane
        k = lax.shift_right_logical(p, 9)
        t = lax.bitwise_and(p, _WW - 1)
        delta = (lax.bitwise_and(k, 1) - lax.shift_right_logical(k, 1)) * _BT
        d = t - (_BT - 1) + delta
        idx = (jnp.clip(d, -_MAX_DIST, _MAX_DIST) + _MAX_DIST) * _NUM_HEADS + h
        idx_v[pl.ds(pl.multiple_of(ch * _L, _L), _L)] = idx
        return _

    lax.fori_loop(0, _PW // _L, build_idx, None)

    for q in range(_PW // 128):
        off = 128 * q
        pltpu.async_copy(
            table_hbm.at[idx_v.at[pl.ds(off, 128)]],
            seg_v.at[pl.ds(off, 128)],
            sem,
        )
    for q in range(_PW // 128):
        pltpu.make_async_copy(
            table_hbm.at[pl.ds(0, 128)], seg_v.at[pl.ds(0, 128)], sem
        ).wait()

    woff = pl.multiple_of(h * _HW + p0, _L)
    pltpu.sync_copy(seg_v, bands_hbm.at[pl.ds(woff, _PW)])


@functools.partial(
    pl.kernel,
    mesh=plsc.VectorSubcoreMesh(core_axis_name="c", subcore_axis_name="s"),
    out_type=jax.ShapeDtypeStruct((_NUM_HEADS * _HW,), jnp.float32),
    scratch_types=[
        pltpu.VMEM((_PW,), jnp.int32),
        pltpu.VMEM((_PW,), jnp.float32),
        pltpu.SemaphoreType.DMA,
    ],
)
def _sc_bands(table_hbm, bands_hbm, idx_v, seg_v, sem):
    _sc_bands_body(table_hbm, bands_hbm, idx_v, seg_v, sem)


def _shear_quarter(w_row, q):
    base = 192 - 64 * q
    wq = w_row[base:base + 320]
    y = jnp.broadcast_to(wq[None, :], (64, 320))
    row = lax.broadcasted_iota(jnp.int32, (64, 320), 0)
    for b in range(6):
        amt = 1 << b
        shifted = jnp.concatenate(
            [y[:, amt:], jnp.zeros((64, amt), jnp.float32)], axis=1
        )
        apply = ((row >> b) & 1) == 0
        y = jnp.where(apply, shifted, y)
    return y[:, :_BT]


def _tc_body(bands_ref, out_ref, scr_ref):
    ib = pl.program_id(1)

    @pl.when(ib == 0)
    def _build():
        for k in range(_NDELTA):
            for q in range(4):
                scr_ref[k, 64 * q:64 * (q + 1), :] = _shear_quarter(
                    bands_ref[0, k, :], q
                )

    c0v = scr_ref[2, 0:1, 0:1]
    c256v = scr_ref[1, 0:1, 255:256]

    for jb in range(_NB):
        val = jnp.where(jb < ib, c0v, c256v)
        out_ref[0, :, jb * _BT:(jb + 1) * _BT] = jnp.broadcast_to(
            val, (_BT, _BT)
        )

    @pl.when(ib >= 1)
    def _left():
        out_ref[0, :, pl.ds((ib - 1) * _BT, _BT)] = scr_ref[2]

    out_ref[0, :, pl.ds(ib * _BT, _BT)] = scr_ref[0]

    @pl.when(ib <= _NB - 2)
    def _right():
        out_ref[0, :, pl.ds((ib + 1) * _BT, _BT)] = scr_ref[1]


@functools.partial(
    pl.pallas_call,
    grid=(_NUM_HEADS, _NB),
    in_specs=[
        pl.BlockSpec((1, _NDELTA, _WW), lambda hh, ib: (hh, 0, 0)),
    ],
    out_specs=pl.BlockSpec((1, _BT, _S), lambda hh, ib: (hh, ib, 0)),
    out_shape=jax.ShapeDtypeStruct((_NUM_HEADS, _S, _S), jnp.float32),
    scratch_shapes=[pltpu.VMEM((_NDELTA, _BT, _BT), jnp.float32)],
)
def _tc_expand(bands_ref, out_ref, scr_ref):
    _tc_body(bands_ref, out_ref, scr_ref)


def kernel(seq_len, table):
    del seq_len
    bands = _sc_bands(table.reshape(-1))
    return _tc_expand(bands.reshape(_NUM_HEADS, _NDELTA, _WW))


# speedup vs baseline: 11.2760x; 1.0405x over previous
import functools

import jax
import jax.numpy as jnp
from jax import lax
from jax.experimental import pallas as pl
from jax.experimental.pallas import tpu as pltpu
from jax.experimental.pallas import tpu_sc as plsc

_NUM_HEADS = 16
_MAX_DIST = 128
_S = 2048
_L = 16
_BT = 256
_NB = _S // _BT
_WW = 2 * _BT
_NDELTA = 3
_HW = _NDELTA * _WW
_PW = _HW // 2


def _sc_bands_body(table_hbm, bands_hbm, idx_v, seg_v, sem):
    h = lax.axis_index("s")
    half = lax.axis_index("c")
    p0 = half * _PW

    lane = lax.iota(jnp.int32, _L)

    def build_idx(ch, _):
        p = p0 + ch * _L + lane
        k = lax.shift_right_logical(p, 9)
        t = lax.bitwise_and(p, _WW - 1)
        delta = (lax.bitwise_and(k, 1) - lax.shift_right_logical(k, 1)) * _BT
        d = t - (_BT - 1) + delta
        idx = (jnp.clip(d, -_MAX_DIST, _MAX_DIST) + _MAX_DIST) * _NUM_HEADS + h
        idx_v[pl.ds(pl.multiple_of(ch * _L, _L), _L)] = idx
        return _

    lax.fori_loop(0, _PW // _L, build_idx, None)

    for q in range(_PW // 128):
        off = 128 * q
        pltpu.async_copy(
            table_hbm.at[idx_v.at[pl.ds(off, 128)]],
            seg_v.at[pl.ds(off, 128)],
            sem,
        )
    for q in range(_PW // 128):
        pltpu.make_async_copy(
            table_hbm.at[pl.ds(0, 128)], seg_v.at[pl.ds(0, 128)], sem
        ).wait()

    woff = pl.multiple_of(h * _HW + p0, _L)
    pltpu.sync_copy(seg_v, bands_hbm.at[pl.ds(woff, _PW)])


@functools.partial(
    pl.kernel,
    mesh=plsc.VectorSubcoreMesh(core_axis_name="c", subcore_axis_name="s"),
    out_type=jax.ShapeDtypeStruct((_NUM_HEADS * _HW,), jnp.float32),
    scratch_types=[
        pltpu.VMEM((_PW,), jnp.int32),
        pltpu.VMEM((_PW,), jnp.float32),
        pltpu.SemaphoreType.DMA,
    ],
)
def _sc_bands(table_hbm, bands_hbm, idx_v, seg_v, sem):
    _sc_bands_body(table_hbm, bands_hbm, idx_v, seg_v, sem)


def _shear_quarter(w_row, q):
    base = 192 - 64 * q
    wq = w_row[base:base + 320]
    y = jnp.broadcast_to(wq[None, :], (64, 320))
    row = lax.broadcasted_iota(jnp.int32, (64, 320), 0)
    for b in range(6):
        amt = 1 << b
        shifted = jnp.concatenate(
            [y[:, amt:], jnp.zeros((64, amt), jnp.float32)], axis=1
        )
        apply = ((row >> b) & 1) == 0
        y = jnp.where(apply, shifted, y)
    return y[:, :_BT]


def _tc_body(bands_ref, out_ref, scr_ref):
    hh = pl.program_id(0)
    ib = pl.program_id(1)
    p = lax.rem(hh, 2)

    @pl.when((hh == 0) & (ib == 0))
    def _build_cold():
        for k in range(_NDELTA):
            for q in range(4):
                scr_ref[0, k, 64 * q:64 * (q + 1), :] = _shear_quarter(
                    bands_ref[0, k, :], q
                )

    for k in range(_NDELTA):
        @pl.when((ib == k + 1) & (hh < _NUM_HEADS - 1))
        def _build_next(k=k):
            for q in range(4):
                scr_ref[1 - p, k, 64 * q:64 * (q + 1), :] = _shear_quarter(
                    bands_ref[0, k, :], q
                )

    c0v = scr_ref[p, 2, 0:1, 0:1]
    c256v = scr_ref[p, 1, 0:1, 255:256]

    for jb in range(_NB):
        @pl.when(jnp.logical_or(jb < ib - 1, jb > ib + 1))
        def _fill(jb=jb):
            val = jnp.where(jb < ib, c0v, c256v)
            out_ref[0, :, jb * _BT:(jb + 1) * _BT] = jnp.broadcast_to(
                val, (_BT, _BT)
            )

    @pl.when(ib >= 1)
    def _left():
        out_ref[0, :, pl.ds((ib - 1) * _BT, _BT)] = scr_ref[p, 2]

    out_ref[0, :, pl.ds(ib * _BT, _BT)] = scr_ref[p, 0]

    @pl.when(ib <= _NB - 2)
    def _right():
        out_ref[0, :, pl.ds((ib + 1) * _BT, _BT)] = scr_ref[p, 1]


@functools.partial(
    pl.pallas_call,
    grid=(_NUM_HEADS, _NB),
    in_specs=[
        pl.BlockSpec(
            (1, _NDELTA, _WW),
            lambda hh, ib: (
                jnp.minimum(hh + (ib >= 1).astype(jnp.int32), _NUM_HEADS - 1),
                0,
                0,
            ),
        ),
    ],
    out_specs=pl.BlockSpec((1, _BT, _S), lambda hh, ib: (hh, ib, 0)),
    out_shape=jax.ShapeDtypeStruct((_NUM_HEADS, _S, _S), jnp.float32),
    scratch_shapes=[pltpu.VMEM((2, _NDELTA, _BT, _BT), jnp.float32)],
)
def _tc_expand(bands_ref, out_ref, scr_ref):
    _tc_body(bands_ref, out_ref, scr_ref)


def kernel(seq_len, table):
    del seq_len
    bands = _sc_bands(table.reshape(-1))
    return _tc_expand(bands.reshape(_NUM_HEADS, _NDELTA, _WW))


# speedup vs baseline: 11.6633x; 1.0343x over previous
import functools

import jax
import jax.numpy as jnp
from jax import lax
from jax.experimental import pallas as pl
from jax.experimental.pallas import tpu as pltpu
from jax.experimental.pallas import tpu_sc as plsc

_NUM_HEADS = 16
_MAX_DIST = 128
_S = 2048
_L = 16
_BT = 256
_NB = _S // _BT
_WW = 2 * _BT
_NDELTA = 3
_HW = _NDELTA * _WW
_PW = _HW // 2


def _sc_bands_body(table_hbm, bands_hbm, idx_v, seg_v, sem):
    h = lax.axis_index("s")
    half = lax.axis_index("c")
    p0 = half * _PW

    lane = lax.iota(jnp.int32, _L)

    def build_idx(ch, _):
        p = p0 + ch * _L + lane
        k = lax.shift_right_logical(p, 9)
        t = lax.bitwise_and(p, _WW - 1)
        delta = (lax.bitwise_and(k, 1) - lax.shift_right_logical(k, 1)) * _BT
        d = t - (_BT - 1) + delta
        idx = (jnp.clip(d, -_MAX_DIST, _MAX_DIST) + _MAX_DIST) * _NUM_HEADS + h
        idx_v[pl.ds(pl.multiple_of(ch * _L, _L), _L)] = idx
        return _

    lax.fori_loop(0, _PW // _L, build_idx, None)

    for q in range(_PW // 128):
        off = 128 * q
        pltpu.async_copy(
            table_hbm.at[idx_v.at[pl.ds(off, 128)]],
            seg_v.at[pl.ds(off, 128)],
            sem,
        )
    for q in range(_PW // 128):
        pltpu.make_async_copy(
            table_hbm.at[pl.ds(0, 128)], seg_v.at[pl.ds(0, 128)], sem
        ).wait()

    woff = pl.multiple_of(h * _HW + p0, _L)
    pltpu.sync_copy(seg_v, bands_hbm.at[pl.ds(woff, _PW)])


@functools.partial(
    pl.kernel,
    mesh=plsc.VectorSubcoreMesh(core_axis_name="c", subcore_axis_name="s"),
    out_type=jax.ShapeDtypeStruct((_NUM_HEADS * _HW,), jnp.float32),
    scratch_types=[
        pltpu.VMEM((_PW,), jnp.int32),
        pltpu.VMEM((_PW,), jnp.float32),
        pltpu.SemaphoreType.DMA,
    ],
)
def _sc_bands(table_hbm, bands_hbm, idx_v, seg_v, sem):
    _sc_bands_body(table_hbm, bands_hbm, idx_v, seg_v, sem)


def _shear_quarter(w_row, q):
    base = 192 - 64 * q
    wq = w_row[base:base + 320]
    y = jnp.broadcast_to(wq[None, :], (64, 320))
    row = lax.broadcasted_iota(jnp.int32, (64, 320), 0)
    for b in range(6):
        amt = 1 << b
        shifted = jnp.concatenate(
            [y[:, amt:], jnp.zeros((64, amt), jnp.float32)], axis=1
        )
        apply = ((row >> b) & 1) == 0
        y = jnp.where(apply, shifted, y)
    return y[:, :_BT]


def _tc_body(bands_ref, out_ref, scr_ref):
    hh = pl.program_id(0)
    g = pl.program_id(1)
    p = lax.rem(hh, 2)

    @pl.when((hh == 0) & (g == 0))
    def _build_cold():
        for k in range(_NDELTA):
            for q in range(4):
                scr_ref[0, k, 64 * q:64 * (q + 1), :] = _shear_quarter(
                    bands_ref[0, k, :], q
                )

    for k in range(_NDELTA):
        @pl.when((g == (1 if k < 2 else 2)) & (hh < _NUM_HEADS - 1))
        def _build_next(k=k):
            for q in range(4):
                scr_ref[1 - p, k, 64 * q:64 * (q + 1), :] = _shear_quarter(
                    bands_ref[0, k, :], q
                )

    c0v = scr_ref[p, 2, 0:1, 0:1]
    c256v = scr_ref[p, 1, 0:1, 255:256]

    for sp in range(2):
        ib = 2 * g + sp
        r0, r1 = sp * _BT, (sp + 1) * _BT

        for jb in range(_NB):
            @pl.when(jnp.logical_or(jb < ib - 1, jb > ib + 1))
            def _fill(jb=jb, r0=r0, r1=r1, ib=ib):
                val = jnp.where(jb < ib, c0v, c256v)
                out_ref[0, r0:r1, jb * _BT:(jb + 1) * _BT] = jnp.broadcast_to(
                    val, (_BT, _BT)
                )

        @pl.when(ib >= 1)
        def _left(r0=r0, r1=r1, ib=ib):
            out_ref[0, r0:r1, pl.ds((ib - 1) * _BT, _BT)] = scr_ref[p, 2]

        out_ref[0, r0:r1, pl.ds(ib * _BT, _BT)] = scr_ref[p, 0]

        @pl.when(ib <= _NB - 2)
        def _right(r0=r0, r1=r1, ib=ib):
            out_ref[0, r0:r1, pl.ds((ib + 1) * _BT, _BT)] = scr_ref[p, 1]


@functools.partial(
    pl.pallas_call,
    grid=(_NUM_HEADS, _NB // 2),
    in_specs=[
        pl.BlockSpec(
            (1, _NDELTA, _WW),
            lambda hh, g: (
                jnp.minimum(hh + (g >= 1).astype(jnp.int32), _NUM_HEADS - 1),
                0,
                0,
            ),
        ),
    ],
    out_specs=pl.BlockSpec((1, 2 * _BT, _S), lambda hh, g: (hh, g, 0)),
    out_shape=jax.ShapeDtypeStruct((_NUM_HEADS, _S, _S), jnp.float32),
    scratch_shapes=[pltpu.VMEM((2, _NDELTA, _BT, _BT), jnp.float32)],
)
def _tc_expand(bands_ref, out_ref, scr_ref):
    _tc_body(bands_ref, out_ref, scr_ref)


def kernel(seq_len, table):
    del seq_len
    bands = _sc_bands(table.reshape(-1))
    return _tc_expand(bands.reshape(_NUM_HEADS, _NDELTA, _WW))


# speedup vs baseline: 12.3719x; 1.0608x over previous
import functools

import jax
import jax.numpy as jnp
from jax import lax
from jax.experimental import pallas as pl
from jax.experimental.pallas import tpu as pltpu
from jax.experimental.pallas import tpu_sc as plsc

_NUM_HEADS = 16
_MAX_DIST = 128
_S = 2048
_L = 16
_BT = 256
_NB = _S // _BT
_WW = 2 * _BT
_NDELTA = 3
_HW = 2 * _WW
_PW = _HW // 2


def _sc_bands_body(table_hbm, bands_hbm, idx_v, seg_v, sem):
    h = lax.axis_index("s")
    half = lax.axis_index("c")
    p0 = half * _PW

    lane = lax.iota(jnp.int32, _L)

    def build_idx(ch, _):
        t = p0 + ch * _L + lane
        d = t - (_WW - 1)
        idx = (jnp.clip(d, -_MAX_DIST, _MAX_DIST) + _MAX_DIST) * _NUM_HEADS + h
        idx_v[pl.ds(pl.multiple_of(ch * _L, _L), _L)] = idx
        return _

    lax.fori_loop(0, _PW // _L, build_idx, None)

    for q in range(_PW // 128):
        off = 128 * q
        pltpu.async_copy(
            table_hbm.at[idx_v.at[pl.ds(off, 128)]],
            seg_v.at[pl.ds(off, 128)],
            sem,
        )
    for q in range(_PW // 128):
        pltpu.make_async_copy(
            table_hbm.at[pl.ds(0, 128)], seg_v.at[pl.ds(0, 128)], sem
        ).wait()

    woff = pl.multiple_of(h * _HW + p0, _L)
    pltpu.sync_copy(seg_v, bands_hbm.at[pl.ds(woff, _PW)])


@functools.partial(
    pl.kernel,
    mesh=plsc.VectorSubcoreMesh(core_axis_name="c", subcore_axis_name="s"),
    out_type=jax.ShapeDtypeStruct((_NUM_HEADS * _HW,), jnp.float32),
    scratch_types=[
        pltpu.VMEM((_PW,), jnp.int32),
        pltpu.VMEM((_PW,), jnp.float32),
        pltpu.SemaphoreType.DMA,
    ],
)
def _sc_bands(table_hbm, bands_hbm, idx_v, seg_v, sem):
    _sc_bands_body(table_hbm, bands_hbm, idx_v, seg_v, sem)


_KOFF = (256, 512, 0)


def _shear_quarter(w_row, q, k):
    base = _KOFF[k] + 192 - 64 * q
    wq = w_row[base:base + 320]
    y = jnp.broadcast_to(wq[None, :], (64, 320))
    row = lax.broadcasted_iota(jnp.int32, (64, 320), 0)
    for b in range(6):
        amt = 1 << b
        shifted = jnp.concatenate(
            [y[:, amt:], jnp.zeros((64, amt), jnp.float32)], axis=1
        )
        apply = ((row >> b) & 1) == 0
        y = jnp.where(apply, shifted, y)
    return y[:, :_BT]


def _tc_body(bands_ref, out_ref, scr_ref):
    hh = pl.program_id(0)
    g = pl.program_id(1)
    p = lax.rem(hh, 2)

    @pl.when((hh == 0) & (g == 0))
    def _build_cold():
        for k in range(_NDELTA):
            for q in range(4):
                scr_ref[0, k, 64 * q:64 * (q + 1), :] = _shear_quarter(
                    bands_ref[0, 0], q, k
                )

    for k in range(_NDELTA):
        @pl.when((g == (1 if k < 2 else 2)) & (hh < _NUM_HEADS - 1))
        def _build_next(k=k):
            for q in range(4):
                scr_ref[1 - p, k, 64 * q:64 * (q + 1), :] = _shear_quarter(
                    bands_ref[0, 0], q, k
                )

    c0v = scr_ref[p, 2, 0:1, 0:1]
    c256v = scr_ref[p, 1, 0:1, 255:256]

    for sp in range(2):
        ib = 2 * g + sp
        r0, r1 = sp * _BT, (sp + 1) * _BT

        for jb in range(_NB):
            @pl.when(jnp.logical_or(jb < ib - 1, jb > ib + 1))
            def _fill(jb=jb, r0=r0, r1=r1, ib=ib):
                val = jnp.where(jb < ib, c0v, c256v)
                out_ref[0, r0:r1, jb * _BT:(jb + 1) * _BT] = jnp.broadcast_to(
                    val, (_BT, _BT)
                )

        @pl.when(ib >= 1)
        def _left(r0=r0, r1=r1, ib=ib):
            out_ref[0, r0:r1, pl.ds((ib - 1) * _BT, _BT)] = scr_ref[p, 2]

        out_ref[0, r0:r1, pl.ds(ib * _BT, _BT)] = scr_ref[p, 0]

        @pl.when(ib <= _NB - 2)
        def _right(r0=r0, r1=r1, ib=ib):
            out_ref[0, r0:r1, pl.ds((ib + 1) * _BT, _BT)] = scr_ref[p, 1]


@functools.partial(
    pl.pallas_call,
    grid=(_NUM_HEADS, _NB // 2),
    in_specs=[
        pl.BlockSpec(
            (1, 1, _HW),
            lambda hh, g: (
                jnp.minimum(hh + (g >= 1).astype(jnp.int32), _NUM_HEADS - 1),
                0,
                0,
            ),
        ),
    ],
    out_specs=pl.BlockSpec((1, 2 * _BT, _S), lambda hh, g: (hh, g, 0)),
    out_shape=jax.ShapeDtypeStruct((_NUM_HEADS, _S, _S), jnp.float32),
    scratch_shapes=[pltpu.VMEM((2, _NDELTA, _BT, _BT), jnp.float32)],
)
def _tc_expand(bands_ref, out_ref, scr_ref):
    _tc_body(bands_ref, out_ref, scr_ref)


def kernel(seq_len, table):
    del seq_len
    bands = _sc_bands(table.reshape(-1))
    return _tc_expand(bands.reshape(_NUM_HEADS, 1, _HW))
